# baseline (speedup 1.0000x reference)
import functools

import jax
import jax.numpy as jnp
from jax import lax
from jax.experimental import pallas as pl
from jax.experimental.pallas import tpu as pltpu
from jax.experimental.pallas import tpu_sc as plsc

P = 16
B = 32
H = 512
W = 512
C = 3
NH = H // P
NW = W // P
N = NH * NW
PR = P * P * C
L = 16
CH = 128
NC = N // CH
QW = 128
QP = QW // P
NQS = NH * 4
MS = 17


def _sc_patch_compact(x):
    @functools.partial(
        pl.kernel,
        mesh=plsc.VectorSubcoreMesh(core_axis_name="c", subcore_axis_name="s"),
        out_type=jax.ShapeDtypeStruct((B, PR, N), jnp.float32),
        scratch_types=[
            pltpu.VMEM((2, C, P, QW), jnp.float32),
            pltpu.VMEM((PR, CH), jnp.float32),
            pltpu.VMEM((PR * MS,), jnp.float32),
            pltpu.SemaphoreType.DMA((2,)),
        ],
        compiler_params=pltpu.CompilerParams(needs_layout_passes=False),
    )
    def k(img, y, bufs, stg, mini, sems):
        b = lax.axis_index("s") * 2 + lax.axis_index("c")
        qv = lax.iota(jnp.int32, L)
        qvm = qv * MS
        zv = jnp.zeros((L,), jnp.float32)

        def src(hh):
            i = hh // 4
            h = hh % 4
            return img.at[b, :, pl.ds(i * P, P), pl.ds(h * QW, QW)]

        def repack(l0):
            def rep_body(o, carry):
                for kk in range(L):
                    row = o * L + kk
                    v2 = plsc.load_gather(mini, [qv + row * MS])
                    stg[row, pl.ds(l0, L)] = v2
                return carry

            lax.fori_loop(0, PR // L, rep_body, 0)

        pltpu.async_copy(src(0), bufs.at[0], sems.at[0])

        def quarter_body(hh, cnt0):
            bi = hh % 2
            pltpu.make_async_copy(src(hh), bufs.at[bi], sems.at[bi]).wait()

            @pl.when(hh + 1 < NQS)
            def _():
                nb = (hh + 1) % 2
                pltpu.async_copy(src(hh + 1), bufs.at[nb], sems.at[nb])

            def patch_body(j, cnt):
                col = qvm + cnt % L
                accs = []
                for c in range(C):
                    acc = jnp.full((L,), -1.0, jnp.float32)
                    for r in range(P):
                        v = bufs[bi, c, r, pl.ds(j * P, L)]
                        acc = jnp.maximum(acc, v)
                        plsc.store_scatter(
                            mini, [col + (r * C + c) * P * MS], v
                        )
                    accs.append(acc)
                m = jnp.maximum(jnp.maximum(accs[0], accs[1]), accs[2])
                keep = jnp.any(m > 0.0).astype(jnp.int32)
                cnt1 = cnt + keep

                @pl.when((keep == 1) & (cnt1 % L == 0))
                def _():
                    repack((cnt1 - L) % CH)

                    @pl.when(cnt1 % CH == 0)
                    def _():
                        pltpu.sync_copy(
                            stg,
                            y.at[b, :, pl.ds((cnt1 // CH - 1) * CH, CH)],
                        )

                return cnt1

            return lax.fori_loop(0, QP, patch_body, cnt0)

        count = lax.fori_loop(0, NQS, quarter_body, jnp.int32(0))

        fill = count % CH
        f16 = count % L
        mfull = count // CH
        fillp = fill - f16 + jnp.where(f16 > 0, L, 0)

        def zero_grp(g, carry):
            def zg_body(o, carry2):
                for kk in range(L):
                    stg[o * L + kk, pl.ds(g * L, L)] = zv
                return carry2

            lax.fori_loop(0, PR // L, zg_body, 0)
            return carry

        def tail_body(m2, carry):
            pltpu.sync_copy(stg, y.at[b, :, pl.ds(m2 * CH, CH)])
            return carry

        @pl.when(count < N)
        def _():
            @pl.when(f16 > 0)
            def _():
                def pad_col(l2, carry):
                    def pc_body(o, carry2):
                        plsc.store_scatter(
                            mini, [qvm + (o * L * MS + l2)], zv
                        )
                        return carry2

                    lax.fori_loop(0, PR // L, pc_body, 0)
                    return carry

                lax.fori_loop(f16, L, pad_col, 0)
                repack(fill - f16)

            lax.fori_loop(fillp // L, CH // L, zero_grp, 0)

            @pl.when(fillp > 0)
            def _():
                pltpu.sync_copy(stg, y.at[b, :, pl.ds(mfull * CH, CH)])

            nfl = mfull + (fillp > 0).astype(jnp.int32)

            @pl.when(nfl < NC)
            def _():
                lax.fori_loop(0, fillp // L, zero_grp, 0)
                lax.fori_loop(nfl, NC, tail_body, 0)

    return k(x)


def kernel(images):
    x = images.transpose(0, 3, 1, 2)
    y = _sc_patch_compact(x)
    return y.reshape(B, P, C, P, N).transpose(0, 4, 1, 3, 2)


# speedup vs baseline: 1.0374x; 1.0374x over previous
import functools

import jax
import jax.numpy as jnp
from jax import lax
from jax.experimental import pallas as pl
from jax.experimental.pallas import tpu as pltpu
from jax.experimental.pallas import tpu_sc as plsc

P = 16
B = 32
H = 512
W = 512
C = 3
NH = H // P
NW = W // P
N = NH * NW
PR = P * P * C
L = 16
CH = 128
NC = N // CH
HW = W // 2
HP = NW // 2
NHS = NH * 2


def _sc_patch_compact(x):
    @functools.partial(
        pl.kernel,
        mesh=plsc.VectorSubcoreMesh(core_axis_name="c", subcore_axis_name="s"),
        out_type=jax.ShapeDtypeStruct((B, PR, N), jnp.float32),
        scratch_types=[
            pltpu.VMEM((2, C, P, HW), jnp.float32),
            pltpu.VMEM((PR, CH), jnp.float32),
            pltpu.SemaphoreType.DMA((2,)),
        ],
        compiler_params=pltpu.CompilerParams(needs_layout_passes=False),
    )
    def k(img, y, bufs, stg, sems):
        b = lax.axis_index("s") * 2 + lax.axis_index("c")
        qv = lax.iota(jnp.int32, L)
        zv = jnp.zeros((L,), jnp.float32)
        rows = [qv + (r * C + c) * P for c in range(C) for r in range(P)]

        def src(hh):
            i = hh // 2
            h = hh % 2
            return img.at[b, :, pl.ds(i * P, P), pl.ds(h * HW, HW)]

        pltpu.async_copy(src(0), bufs.at[0], sems.at[0])

        def half_body(hh, cnt0):
            bi = hh % 2
            pltpu.make_async_copy(src(hh), bufs.at[bi], sems.at[bi]).wait()

            @pl.when(hh + 1 < NHS)
            def _():
                nb = (hh + 1) % 2
                pltpu.async_copy(src(hh + 1), bufs.at[nb], sems.at[nb])

            def patch_body(j, cnt):
                lane = jnp.full((L,), cnt % CH, jnp.int32)
                accs = []
                for c in range(C):
                    acc = jnp.full((L,), -1.0, jnp.float32)
                    for r in range(P):
                        v = bufs[bi, c, r, pl.ds(j * P, L)]
                        acc = jnp.maximum(acc, v)
                        plsc.store_scatter(
                            stg, [rows[c * P + r], lane], v
                        )
                    accs.append(acc)
                m = jnp.maximum(jnp.maximum(accs[0], accs[1]), accs[2])
                pv = plsc.all_reduce_population_count(m > 0.0)
                keep = (pv[0] > 0).astype(jnp.int32)
                cnt1 = cnt + keep

                @pl.when((keep == 1) & (cnt1 % CH == 0))
                def _():
                    pltpu.sync_copy(
                        stg, y.at[b, :, pl.ds((cnt1 // CH - 1) * CH, CH)]
                    )

                return cnt1

            return lax.fori_loop(0, HP, patch_body, cnt0)

        count = lax.fori_loop(0, NHS, half_body, jnp.int32(0))

        fill = count % CH
        mfull = count // CH

        def zero_lanes(l2, carry):
            lane = jnp.full((L,), l2, jnp.int32)
            for row in range(PR // L):
                plsc.store_scatter(stg, [qv + row * L, lane], zv)
            return carry

        def tail_body(m2, carry):
            pltpu.sync_copy(stg, y.at[b, :, pl.ds(m2 * CH, CH)])
            return carry

        @pl.when(count < N)
        def _():
            lax.fori_loop(fill, CH, zero_lanes, 0)

            @pl.when(fill > 0)
            def _():
                pltpu.sync_copy(stg, y.at[b, :, pl.ds(mfull * CH, CH)])

            nfl = mfull + (fill > 0).astype(jnp.int32)

            @pl.when(nfl < NC)
            def _():
                lax.fori_loop(0, fill, zero_lanes, 0)
                lax.fori_loop(nfl, NC, tail_body, 0)

    return k(x)


def kernel(images):
    x = images.transpose(0, 3, 1, 2)
    y = _sc_patch_compact(x)
    return y.reshape(B, P, C, P, N).transpose(0, 4, 1, 3, 2)


# speedup vs baseline: 1.6539x; 1.5943x over previous
import functools

import jax
import jax.numpy as jnp
from jax import lax
from jax.experimental import pallas as pl
from jax.experimental.pallas import tpu as pltpu
from jax.experimental.pallas import tpu_sc as plsc

P = 16
B = 32
H = 512
W = 512
C = 3
NH = H // P
NW = W // P
N = NH * NW
PR = P * P * C
PZ = PR
L = 16
CH = 128
NC = N // CH
QW = 128
QP = QW // P
NQS = NH * 4
NB = PR // L


def _sc_patch_compact(x):
    @functools.partial(
        pl.kernel,
        mesh=plsc.VectorSubcoreMesh(core_axis_name="c", subcore_axis_name="s"),
        out_type=jax.ShapeDtypeStruct((B, PR, N), jnp.float32),
        scratch_types=[
            pltpu.VMEM((2, C, P, QW), jnp.float32),
            pltpu.VMEM((PR, CH), jnp.float32),
            pltpu.VMEM((L, PZ), jnp.float32),
            pltpu.SemaphoreType.DMA((2,)),
        ],
        compiler_params=pltpu.CompilerParams(needs_layout_passes=False),
    )
    def k(img, y, bufs, stg, pbuf, sems):
        b = lax.axis_index("s") * 2 + lax.axis_index("c")
        qv = lax.iota(jnp.int32, L)
        zv = jnp.zeros((L,), jnp.float32)
        perms = [qv ^ (1 << s) for s in range(4)]
        masks = [(qv & (1 << s)) != 0 for s in range(4)]
        gd = lax.GatherDimensionNumbers(
            offset_dims=(), collapsed_slice_dims=(0,), start_index_map=(0,)
        )

        def lperm(vx, idx):
            return lax.gather(
                vx, idx[:, None], gd, slice_sizes=(1,),
                mode=lax.GatherScatterMode.PROMISE_IN_BOUNDS,
            )

        def src(hh):
            return img.at[
                b, :, pl.ds((hh // 4) * P, P), pl.ds((hh % 4) * QW, QW)
            ]

        def group_flush(l0):
            def blk_body(rc, carry):
                w = [pbuf[i, pl.ds(rc * L, L)] for i in range(L)]
                for s in range(4):
                    kk = 1 << s
                    nw = list(w)
                    for i in range(L):
                        if i & kk == 0:
                            a, bb = w[i], w[i | kk]
                            bx = lperm(bb, perms[s])
                            ax = lperm(a, perms[s])
                            nw[i] = jnp.where(masks[s], bx, a)
                            nw[i | kk] = jnp.where(masks[s], bb, ax)
                    w = nw
                for q in range(L):
                    stg[rc * L + q, pl.ds(l0, L)] = w[q]
                return carry

            lax.fori_loop(0, NB, blk_body, 0)

        pltpu.async_copy(src(0), bufs.at[0], sems.at[0])

        def quarter_body(hh, cnt0):
            bi = hh % 2
            pltpu.make_async_copy(src(hh), bufs.at[bi], sems.at[bi]).wait()

            @pl.when(hh + 1 < NQS)
            def _():
                nb = (hh + 1) % 2
                pltpu.async_copy(src(hh + 1), bufs.at[nb], sems.at[nb])

            def patch_body(j, cnt):
                lg = cnt % L
                accs = []
                for c in range(C):
                    acc = jnp.full((L,), -1.0, jnp.float32)
                    for r in range(P):
                        v = bufs[bi, c, r, pl.ds(j * P, L)]
                        acc = jnp.maximum(acc, v)
                        pbuf[lg, pl.ds((r * C + c) * P, L)] = v
                    accs.append(acc)
                m = jnp.maximum(jnp.maximum(accs[0], accs[1]), accs[2])
                pv = plsc.all_reduce_population_count(m > 0.0)
                keep = (pv[0] > 0).astype(jnp.int32)
                cnt1 = cnt + keep

                @pl.when((keep == 1) & (cnt1 % L == 0))
                def _():
                    group_flush((cnt1 - L) % CH)

                    @pl.when(cnt1 % CH == 0)
                    def _():
                        pltpu.sync_copy(
                            stg,
                            y.at[b, :, pl.ds((cnt1 // CH - 1) * CH, CH)],
                        )

                return cnt1

            return lax.fori_loop(0, QP, patch_body, cnt0)

        count = lax.fori_loop(0, NQS, quarter_body, jnp.int32(0))

        fill = count % CH
        f16 = count % L
        mfull = count // CH
        fillp = fill - f16 + jnp.where(f16 > 0, L, 0)

        def zero_grp(g, carry):
            def zg_body(o, carry2):
                for kk in range(L):
                    stg[o * L + kk, pl.ds(g * L, L)] = zv
                return carry2

            lax.fori_loop(0, NB, zg_body, 0)
            return carry

        def tail_body(m2, carry):
            pltpu.sync_copy(stg, y.at[b, :, pl.ds(m2 * CH, CH)])
            return carry

        @pl.when(count < N)
        def _():
            @pl.when(f16 > 0)
            def _():
                def pad_row(l2, carry):
                    for rc in range(NB):
                        pbuf[l2, pl.ds(rc * L, L)] = zv
                    return carry

                lax.fori_loop(f16, L, pad_row, 0)
                group_flush(fill - f16)

            lax.fori_loop(fillp // L, CH // L, zero_grp, 0)

            @pl.when(fillp > 0)
            def _():
                pltpu.sync_copy(stg, y.at[b, :, pl.ds(mfull * CH, CH)])

            nfl = mfull + (fillp > 0).astype(jnp.int32)

            @pl.when(nfl < NC)
            def _():
                lax.fori_loop(0, fillp // L, zero_grp, 0)
                lax.fori_loop(nfl, NC, tail_body, 0)

    return k(x)


def kernel(images):
    x = images.transpose(0, 3, 1, 2)
    y = _sc_patch_compact(x)
    return y.reshape(B, P, C, P, N).transpose(0, 4, 1, 3, 2)
